# baseline (speedup 1.0000x reference)
import jax
import jax.numpy as jnp
from jax import lax
from jax.experimental import pallas as pl
from jax.experimental.pallas import tpu as pltpu
from jax.experimental.pallas import tpu_sc as plsc

_B = 4096
_D = 64
_NC = 2
_NS = 16
_NW = _NC * _NS
_BPW = _B // _NW
_NG = _BPW // 16

_LN2 = 0.6931471805599453
_LN100 = 4.605170185988092
_LOG2_POLY = (
    0.014778755424481588,
    -0.07684890405801897,
    0.1904211707113626,
    -0.32311624947178846,
    0.4724996763418957,
    -0.7203866484224759,
    1.4426521148584406,
    3.1958385927744075e-07,
)


def _ln(x):
    bits = plsc.bitcast(x, jnp.int32)
    e = ((bits >> 23) & 0xFF) - 127
    m = plsc.bitcast((bits & 0x007FFFFF) | 0x3F800000, jnp.float32)
    t = m - 1.0
    p = jnp.full((16,), _LOG2_POLY[0], dtype=jnp.float32)
    for coef in _LOG2_POLY[1:]:
        p = p * t + coef
    return (e.astype(jnp.float32) + p) * _LN2


def _sc_body(c_data, p_data, labels, c_table, c_bias, p_table, p_bias,
             out, cidx_v, pidx_v, lam_v, crows_v, prows_v, cb_v, pb_v,
             pvec_v, allv_v, outv_v, shared, sems):
    cid = lax.axis_index("c")
    sid = lax.axis_index("s")
    wid = cid * _NS + sid
    base = wid * _BPW

    pltpu.sync_copy(c_data.at[pl.ds(base, _BPW)], cidx_v)
    pltpu.sync_copy(p_data.at[pl.ds(base, _BPW)], pidx_v)
    cps = [
        pltpu.async_copy(c_table.at[cidx_v], crows_v, sems.at[0]),
        pltpu.async_copy(p_table.at[pidx_v], prows_v, sems.at[1]),
        pltpu.async_copy(c_bias.at[cidx_v], cb_v, sems.at[2]),
        pltpu.async_copy(p_bias.at[pidx_v], pb_v, sems.at[3]),
        pltpu.async_copy(labels.at[pl.ds(base, _BPW)], lam_v, sems.at[4]),
    ]
    for cp in cps:
        cp.wait()

    lane = lax.iota(jnp.int32, 16)
    zero16 = jnp.zeros((16,), jnp.int32)

    acc_w = jnp.zeros((16,), jnp.float32)
    acc_a = jnp.zeros((16,), jnp.float32)
    acc_a2 = jnp.zeros((16,), jnp.float32)
    s_wb = jnp.float32(0.0)
    s_wb2 = jnp.float32(0.0)
    for g in range(_NG):
        lam = lam_v[pl.ds(g * 16, 16)]
        lnl = _ln(lam)
        w = jnp.minimum(jnp.exp(0.75 * (lnl - _LN100)), 1.0)
        acc_w += w
        rows = lane + (g * 16)
        a16 = (plsc.load_gather(cb_v, [rows, zero16])
               + plsc.load_gather(pb_v, [rows, zero16]))
        acc_a += a16
        acc_a2 += a16 * a16
        for jj in range(16):
            j = g * 16 + jj
            v = crows_v[j, pl.ds(0, 16)] * prows_v[j, pl.ds(0, 16)]
            for kk in range(1, _D // 16):
                v += (crows_v[j, pl.ds(kk * 16, 16)]
                      * prows_v[j, pl.ds(kk * 16, 16)])
            dot = jnp.sum(v)
            bv = dot - lnl[jj]
            wj = w[jj]
            s_wb += wj * bv
            s_wb2 += wj * (bv * bv)
    s_w = jnp.sum(acc_w)
    s_a = jnp.sum(acc_a)
    s_a2 = jnp.sum(acc_a2)

    packed = jnp.where(lane == 0, s_w, 0.0)
    packed = jnp.where(lane == 1, s_wb, packed)
    packed = jnp.where(lane == 2, s_wb2, packed)
    packed = jnp.where(lane == 3, s_a, packed)
    packed = jnp.where(lane == 4, s_a2, packed)
    pvec_v[...] = packed.astype(jnp.float32)

    pltpu.sync_copy(pvec_v, shared.at[sid])
    plsc.subcore_barrier()

    @pl.when(sid == 0)
    def _():
        pltpu.sync_copy(shared, allv_v)
        tot = allv_v[0, :]
        for k in range(1, _NS):
            tot += allv_v[k, :]
        outv_v[...] = tot
        pltpu.sync_copy(outv_v, out.at[cid])


@jax.jit
def kernel(c_data, p_data, labels, c_table, c_bias, p_table, p_bias):
    mesh = plsc.VectorSubcoreMesh(core_axis_name="c", subcore_axis_name="s")
    partials = pl.kernel(
        _sc_body,
        out_type=jax.ShapeDtypeStruct((_NC, 16), jnp.float32),
        mesh=mesh,
        compiler_params=pltpu.CompilerParams(
            needs_layout_passes=False, use_tc_tiling_on_sc=False),
        scratch_types=[
            pltpu.VMEM((_BPW,), jnp.int32),
            pltpu.VMEM((_BPW,), jnp.int32),
            pltpu.VMEM((_BPW,), jnp.float32),
            pltpu.VMEM((_BPW, _D), jnp.float32),
            pltpu.VMEM((_BPW, _D), jnp.float32),
            pltpu.VMEM((_BPW, 1), jnp.float32),
            pltpu.VMEM((_BPW, 1), jnp.float32),
            pltpu.VMEM((16,), jnp.float32),
            pltpu.VMEM((_NS, 16), jnp.float32),
            pltpu.VMEM((16,), jnp.float32),
            pltpu.VMEM_SHARED((_NS, 16), jnp.float32),
            pltpu.SemaphoreType.DMA((5,)),
        ],
    )(c_data, p_data, labels, c_table, c_bias, p_table, p_bias)
    tot = partials[0] + partials[1]
    bf = jnp.float32(_B)
    loss = (tot[0] * tot[4] + 2.0 * tot[3] * tot[1] + bf * tot[2]) / (bf * bf)
    return loss


# speedup vs baseline: 2.2531x; 2.2531x over previous
import jax
import jax.numpy as jnp
from jax import lax
from jax.experimental import pallas as pl
from jax.experimental.pallas import tpu as pltpu
from jax.experimental.pallas import tpu_sc as plsc

_B = 4096
_D = 64
_V = 100000
_NC = 2
_NS = 16
_NW = _NC * _NS
_BPW = _B // _NW
_NG = _BPW // 16

_LN2 = 0.6931471805599453
_LN100 = 4.605170185988092
_LOG2_POLY = (
    0.014778755424481588,
    -0.07684890405801897,
    0.1904211707113626,
    -0.32311624947178846,
    0.4724996763418957,
    -0.7203866484224759,
    1.4426521148584406,
    3.1958385927744075e-07,
)


def _ln(x):
    bits = plsc.bitcast(x, jnp.int32)
    e = ((bits >> 23) & 0xFF) - 127
    m = plsc.bitcast((bits & 0x007FFFFF) | 0x3F800000, jnp.float32)
    t = m - 1.0
    p = jnp.full((16,), _LOG2_POLY[0], dtype=jnp.float32)
    for coef in _LOG2_POLY[1:]:
        p = p * t + coef
    return (e.astype(jnp.float32) + p) * _LN2


def _sc_body(c_data, p_data, labels, c_table, biases, p_table,
             out, cidx_v, pidx_v, pidx2_v, lam_v, crows_v, prows_v,
             cb_v, pb_v, pvec_v, allv_v, outv_v, shared, sems):
    cid = lax.axis_index("c")
    sid = lax.axis_index("s")
    wid = cid * _NS + sid
    base = wid * _BPW

    pltpu.sync_copy(c_data.at[pl.ds(base, _BPW)], cidx_v)
    pltpu.sync_copy(p_data.at[pl.ds(base, _BPW)], pidx_v)
    for g in range(_NG):
        pidx2_v[pl.ds(g * 16, 16)] = pidx_v[pl.ds(g * 16, 16)] + _V
    cps = [
        pltpu.async_copy(c_table.at[cidx_v], crows_v, sems.at[0]),
        pltpu.async_copy(p_table.at[pidx_v], prows_v, sems.at[1]),
        pltpu.async_copy(biases.at[cidx_v], cb_v, sems.at[2]),
        pltpu.async_copy(biases.at[pidx2_v], pb_v, sems.at[3]),
        pltpu.async_copy(labels.at[pl.ds(base, _BPW)], lam_v, sems.at[4]),
    ]
    for cp in cps:
        cp.wait()

    lane = lax.iota(jnp.int32, 16)

    acc_w = jnp.zeros((16,), jnp.float32)
    acc_a = jnp.zeros((16,), jnp.float32)
    acc_a2 = jnp.zeros((16,), jnp.float32)
    s_wb = jnp.float32(0.0)
    s_wb2 = jnp.float32(0.0)
    for g in range(_NG):
        lam = lam_v[pl.ds(g * 16, 16)]
        lnl = _ln(lam)
        w = jnp.minimum(jnp.exp(0.75 * (lnl - _LN100)), 1.0)
        acc_w += w
        a16 = cb_v[pl.ds(g * 16, 16)] + pb_v[pl.ds(g * 16, 16)]
        acc_a += a16
        acc_a2 += a16 * a16
        for jj in range(16):
            j = g * 16 + jj
            v = crows_v[j, pl.ds(0, 16)] * prows_v[j, pl.ds(0, 16)]
            for kk in range(1, _D // 16):
                v += (crows_v[j, pl.ds(kk * 16, 16)]
                      * prows_v[j, pl.ds(kk * 16, 16)])
            dot = jnp.sum(v)
            bv = dot - lnl[jj]
            wj = w[jj]
            s_wb += wj * bv
            s_wb2 += wj * (bv * bv)
    s_w = jnp.sum(acc_w)
    s_a = jnp.sum(acc_a)
    s_a2 = jnp.sum(acc_a2)

    packed = jnp.where(lane == 0, s_w, 0.0)
    packed = jnp.where(lane == 1, s_wb, packed)
    packed = jnp.where(lane == 2, s_wb2, packed)
    packed = jnp.where(lane == 3, s_a, packed)
    packed = jnp.where(lane == 4, s_a2, packed)
    pvec_v[...] = packed.astype(jnp.float32)

    pltpu.sync_copy(pvec_v, shared.at[sid])
    plsc.subcore_barrier()

    @pl.when(sid == 0)
    def _():
        pltpu.sync_copy(shared, allv_v)
        tot = allv_v[0, :]
        for k in range(1, _NS):
            tot += allv_v[k, :]
        outv_v[...] = tot
        pltpu.sync_copy(outv_v, out.at[cid])


@jax.jit
def kernel(c_data, p_data, labels, c_table, c_bias, p_table, p_bias):
    mesh = plsc.VectorSubcoreMesh(core_axis_name="c", subcore_axis_name="s")
    partials = pl.kernel(
        _sc_body,
        out_type=jax.ShapeDtypeStruct((_NC, 16), jnp.float32),
        mesh=mesh,
        compiler_params=pltpu.CompilerParams(
            needs_layout_passes=False, use_tc_tiling_on_sc=False),
        scratch_types=[
            pltpu.VMEM((_BPW,), jnp.int32),
            pltpu.VMEM((_BPW,), jnp.int32),
            pltpu.VMEM((_BPW,), jnp.int32),
            pltpu.VMEM((_BPW,), jnp.float32),
            pltpu.VMEM((_BPW, _D), jnp.float32),
            pltpu.VMEM((_BPW, _D), jnp.float32),
            pltpu.VMEM((_BPW,), jnp.float32),
            pltpu.VMEM((_BPW,), jnp.float32),
            pltpu.VMEM((16,), jnp.float32),
            pltpu.VMEM((_NS, 16), jnp.float32),
            pltpu.VMEM((16,), jnp.float32),
            pltpu.VMEM_SHARED((_NS, 16), jnp.float32),
            pltpu.SemaphoreType.DMA((5,)),
        ],
    )(c_data, p_data, labels, c_table,
      jnp.concatenate([jnp.reshape(c_bias, (-1,)), jnp.reshape(p_bias, (-1,))]),
      p_table)
    tot = partials[0] + partials[1]
    bf = jnp.float32(_B)
    loss = (tot[0] * tot[4] + 2.0 * tot[3] * tot[1] + bf * tot[2]) / (bf * bf)
    return loss


# speedup vs baseline: 4.9098x; 2.1791x over previous
import jax
import jax.numpy as jnp
from jax import lax
from jax.experimental import pallas as pl
from jax.experimental.pallas import tpu as pltpu
from jax.experimental.pallas import tpu_sc as plsc

_B = 4096
_D = 64
_V = 100000
_NC = 2
_NS = 16
_NW = _NC * _NS
_BPW = _B // _NW
_NG = _BPW // 16
_DPW = _D // _NW
_SEC = _B // _NS

_LN2 = 0.6931471805599453
_LN100 = 4.605170185988092
_LOG2_POLY = (
    0.014778755424481588,
    -0.07684890405801897,
    0.1904211707113626,
    -0.32311624947178846,
    0.4724996763418957,
    -0.7203866484224759,
    1.4426521148584406,
    3.1958385927744075e-07,
)


def _ln(x):
    bits = plsc.bitcast(x, jnp.int32)
    e = ((bits >> 23) & 0xFF) - 127
    m = plsc.bitcast((bits & 0x007FFFFF) | 0x3F800000, jnp.float32)
    t = m - 1.0
    p = jnp.full((16,), _LOG2_POLY[0], dtype=jnp.float32)
    for coef in _LOG2_POLY[1:]:
        p = p * t + coef
    return (e.astype(jnp.float32) + p) * _LN2


def _dots_body(c_data, p_data, c_tt, p_tt, out, cidx_v, pidx_v, row_v,
               cval_v, dot_v, tot_v, shared, sems):
    cid = lax.axis_index("c")
    sid = lax.axis_index("s")
    wid = cid * _NS + sid

    pltpu.sync_copy(c_data, cidx_v)
    pltpu.sync_copy(p_data, pidx_v)

    zero = jnp.zeros((16,), jnp.float32)
    for k in range(_B // 16):
        dot_v[pl.ds(k * 16, 16)] = zero

    for dd in range(_DPW):
        d = wid * _DPW + dd

        pltpu.async_copy(c_tt.at[d], row_v, sems.at[0]).wait()

        def cgather(k, _):
            iv = cidx_v[pl.ds(k * 16, 16)]
            cval_v[pl.ds(k * 16, 16)] = plsc.load_gather(row_v, [iv])
            return 0

        lax.fori_loop(0, _B // 16, cgather, 0)

        pltpu.async_copy(p_tt.at[d], row_v, sems.at[0]).wait()

        def pacc(k, _):
            iv = pidx_v[pl.ds(k * 16, 16)]
            pv = plsc.load_gather(row_v, [iv])
            dot_v[pl.ds(k * 16, 16)] += cval_v[pl.ds(k * 16, 16)] * pv
            return 0

        lax.fori_loop(0, _B // 16, pacc, 0)

    pltpu.sync_copy(dot_v, shared.at[sid])
    plsc.subcore_barrier()

    base = sid * _SEC
    for k in range(_SEC // 16):
        tot_v[pl.ds(k * 16, 16)] = zero
    for r in range(_NS):
        pltpu.sync_copy(shared.at[r, pl.ds(base, _SEC)], cval_v.at[pl.ds(0, _SEC)])
        for k in range(_SEC // 16):
            tot_v[pl.ds(k * 16, 16)] += cval_v[pl.ds(k * 16, 16)]
    pltpu.sync_copy(tot_v, out.at[pl.ds(cid * _B + base, _SEC)])


def _loss_body(dots, labels, biases, c_data, p_data, out, cidx_v, pidx_v,
               pidx2_v, lam_v, dot_v, cb_v, pb_v, pvec_v, allv_v, packv_v,
               outv_v, shared, sems):
    cid = lax.axis_index("c")
    sid = lax.axis_index("s")
    wid = cid * _NS + sid
    base = wid * _BPW

    pltpu.sync_copy(c_data.at[pl.ds(base, _BPW)], cidx_v)
    pltpu.sync_copy(p_data.at[pl.ds(base, _BPW)], pidx_v)
    for g in range(_NG):
        pidx2_v[pl.ds(g * 16, 16)] = pidx_v[pl.ds(g * 16, 16)] + _V
    cps = [
        pltpu.async_copy(biases.at[cidx_v], cb_v, sems.at[0]),
        pltpu.async_copy(biases.at[pidx2_v], pb_v, sems.at[1]),
        pltpu.async_copy(labels.at[pl.ds(base, _BPW)], lam_v, sems.at[2]),
        pltpu.async_copy(dots.at[pl.ds(base, _BPW)], dot_v, sems.at[3]),
        pltpu.async_copy(dots.at[pl.ds(_B + base, _BPW)], pvec_v, sems.at[4]),
    ]
    for cp in cps:
        cp.wait()

    lane = lax.iota(jnp.int32, 16)

    acc_w = jnp.zeros((16,), jnp.float32)
    acc_a = jnp.zeros((16,), jnp.float32)
    acc_a2 = jnp.zeros((16,), jnp.float32)
    s_wb = jnp.float32(0.0)
    s_wb2 = jnp.float32(0.0)
    for g in range(_NG):
        lam = lam_v[pl.ds(g * 16, 16)]
        lnl = _ln(lam)
        w = jnp.minimum(jnp.exp(0.75 * (lnl - _LN100)), 1.0)
        acc_w += w
        a16 = cb_v[pl.ds(g * 16, 16)] + pb_v[pl.ds(g * 16, 16)]
        acc_a += a16
        acc_a2 += a16 * a16
        dot16 = dot_v[pl.ds(g * 16, 16)] + pvec_v[pl.ds(g * 16, 16)]
        bv16 = dot16 - lnl
        wb = w * bv16
        s_wb += jnp.sum(wb)
        s_wb2 += jnp.sum(wb * bv16)
    s_w = jnp.sum(acc_w)
    s_a = jnp.sum(acc_a)
    s_a2 = jnp.sum(acc_a2)

    packed = jnp.where(lane == 0, s_w, 0.0)
    packed = jnp.where(lane == 1, s_wb, packed)
    packed = jnp.where(lane == 2, s_wb2, packed)
    packed = jnp.where(lane == 3, s_a, packed)
    packed = jnp.where(lane == 4, s_a2, packed)
    packv_v[...] = packed.astype(jnp.float32)

    pltpu.sync_copy(packv_v, shared.at[pl.ds(sid * 16, 16)])
    plsc.subcore_barrier()

    @pl.when(sid == 0)
    def _():
        pltpu.sync_copy(shared, allv_v)
        tot = allv_v[pl.ds(0, 16)]
        for k in range(1, _NS):
            tot += allv_v[pl.ds(k * 16, 16)]
        outv_v[...] = tot
        pltpu.sync_copy(outv_v, out.at[pl.ds(cid * 16, 16)])


_MESH = plsc.VectorSubcoreMesh(core_axis_name="c", subcore_axis_name="s")
_PARAMS = pltpu.CompilerParams(needs_layout_passes=False)


@jax.jit
def kernel(c_data, p_data, labels, c_table, c_bias, p_table, p_bias):
    dots = pl.kernel(
        _dots_body,
        out_type=jax.ShapeDtypeStruct((_NC * _B,), jnp.float32),
        mesh=_MESH,
        compiler_params=_PARAMS,
        scratch_types=[
            pltpu.VMEM((_B,), jnp.int32),
            pltpu.VMEM((_B,), jnp.int32),
            pltpu.VMEM((_V,), jnp.float32),
            pltpu.VMEM((_B,), jnp.float32),
            pltpu.VMEM((_B,), jnp.float32),
            pltpu.VMEM((_SEC,), jnp.float32),
            pltpu.VMEM_SHARED((_NS, _B), jnp.float32),
            pltpu.SemaphoreType.DMA((1,)),
        ],
    )(c_data, p_data, c_table.T, p_table.T)

    partials = pl.kernel(
        _loss_body,
        out_type=jax.ShapeDtypeStruct((_NC * 16,), jnp.float32),
        mesh=_MESH,
        compiler_params=_PARAMS,
        scratch_types=[
            pltpu.VMEM((_BPW,), jnp.int32),
            pltpu.VMEM((_BPW,), jnp.int32),
            pltpu.VMEM((_BPW,), jnp.int32),
            pltpu.VMEM((_BPW,), jnp.float32),
            pltpu.VMEM((_BPW,), jnp.float32),
            pltpu.VMEM((_BPW,), jnp.float32),
            pltpu.VMEM((_BPW,), jnp.float32),
            pltpu.VMEM((_BPW,), jnp.float32),
            pltpu.VMEM((_NS * 16,), jnp.float32),
            pltpu.VMEM((16,), jnp.float32),
            pltpu.VMEM((16,), jnp.float32),
            pltpu.VMEM_SHARED((_NS * 16,), jnp.float32),
            pltpu.SemaphoreType.DMA((5,)),
        ],
    )(dots, labels,
      jnp.concatenate([jnp.reshape(c_bias, (-1,)), jnp.reshape(p_bias, (-1,))]),
      c_data, p_data)

    tot = partials[:16] + partials[16:]
    bf = jnp.float32(_B)
    loss = (tot[0] * tot[4] + 2.0 * tot[3] * tot[1] + bf * tot[2]) / (bf * bf)
    return loss


# speedup vs baseline: 5.1180x; 1.0424x over previous
import jax
import jax.numpy as jnp
from jax import lax
from jax.experimental import pallas as pl
from jax.experimental.pallas import tpu as pltpu
from jax.experimental.pallas import tpu_sc as plsc

_B = 4096
_D = 64
_V = 100000
_NC = 2
_NS = 16
_NW = _NC * _NS
_BPW = _B // _NW
_NG = _BPW // 16
_DPW = _D // _NW
_SEC = _B // _NS
_H0 = 49920
_H1 = _V - _H0

_LN2 = 0.6931471805599453
_LN100 = 4.605170185988092
_LOG2_POLY = (
    0.014778755424481588,
    -0.07684890405801897,
    0.1904211707113626,
    -0.32311624947178846,
    0.4724996763418957,
    -0.7203866484224759,
    1.4426521148584406,
    3.1958385927744075e-07,
)


def _ln(x):
    bits = plsc.bitcast(x, jnp.int32)
    e = ((bits >> 23) & 0xFF) - 127
    m = plsc.bitcast((bits & 0x007FFFFF) | 0x3F800000, jnp.float32)
    t = m - 1.0
    p = jnp.full((16,), _LOG2_POLY[0], dtype=jnp.float32)
    for coef in _LOG2_POLY[1:]:
        p = p * t + coef
    return (e.astype(jnp.float32) + p) * _LN2


def _dots_body(c_data, p_data, c_tt, p_tt, out, cidx_v, pidx_v, rowa_v,
               rowb_v, cval_v, dot_v, tot_v, shared, sems):
    cid = lax.axis_index("c")
    sid = lax.axis_index("s")
    wid = cid * _NS + sid

    pltpu.sync_copy(c_data, cidx_v)
    pltpu.sync_copy(p_data, pidx_v)

    zero = jnp.zeros((16,), jnp.float32)
    for k in range(_B // 16):
        dot_v[pl.ds(k * 16, 16)] = zero

    bufs = (rowa_v, rowb_v)
    tasks = []
    for dd in range(_DPW):
        d = wid * _DPW + dd
        for tbl, idxv, kind0, kind1 in ((c_tt, cidx_v, "cw", "ca"),
                                        (p_tt, pidx_v, "p", "p")):
            tasks.append((tbl, d, 0, idxv, kind0))
            tasks.append((tbl, d, 1, idxv, kind1))

    def start(k):
        tbl, d, h, _, _ = tasks[k]
        lo, ln = (0, _H0) if h == 0 else (_H0, _H1)
        return pltpu.async_copy(tbl.at[d, pl.ds(lo, ln)],
                                bufs[k % 2].at[pl.ds(0, ln)],
                                sems.at[k % 2])

    def process(k):
        _, _, h, idxv, kind = tasks[k]
        lo, ln = (0, _H0) if h == 0 else (_H0, _H1)
        buf = bufs[k % 2]

        def body(j, _):
            iv = idxv[pl.ds(j * 16, 16)] - lo
            mask = (iv >= 0) & (iv < ln)
            ivc = jnp.minimum(jnp.maximum(iv, 0), ln - 1)
            g = jnp.where(mask, plsc.load_gather(buf, [ivc]), 0.0)
            if kind == "cw":
                cval_v[pl.ds(j * 16, 16)] = g
            elif kind == "ca":
                cval_v[pl.ds(j * 16, 16)] += g
            else:
                dot_v[pl.ds(j * 16, 16)] += cval_v[pl.ds(j * 16, 16)] * g
            return 0

        lax.fori_loop(0, _B // 16, body, 0)

    cps = {0: start(0)}
    for k in range(len(tasks)):
        if k + 1 < len(tasks):
            cps[k + 1] = start(k + 1)
        cps[k].wait()
        process(k)

    pltpu.sync_copy(dot_v, shared.at[sid])
    plsc.subcore_barrier()

    base = sid * _SEC
    for k in range(_SEC // 16):
        tot_v[pl.ds(k * 16, 16)] = zero
    for r in range(_NS):
        pltpu.sync_copy(shared.at[r, pl.ds(base, _SEC)], cval_v.at[pl.ds(0, _SEC)])
        for k in range(_SEC // 16):
            tot_v[pl.ds(k * 16, 16)] += cval_v[pl.ds(k * 16, 16)]
    pltpu.sync_copy(tot_v, out.at[pl.ds(cid * _B + base, _SEC)])


def _loss_body(dots, labels, biases, c_data, p_data, out, cidx_v, pidx_v,
               pidx2_v, lam_v, dot_v, cb_v, pb_v, pvec_v, allv_v, packv_v,
               outv_v, shared, sems):
    cid = lax.axis_index("c")
    sid = lax.axis_index("s")
    wid = cid * _NS + sid
    base = wid * _BPW

    pltpu.sync_copy(c_data.at[pl.ds(base, _BPW)], cidx_v)
    pltpu.sync_copy(p_data.at[pl.ds(base, _BPW)], pidx_v)
    for g in range(_NG):
        pidx2_v[pl.ds(g * 16, 16)] = pidx_v[pl.ds(g * 16, 16)] + _V
    cps = [
        pltpu.async_copy(biases.at[cidx_v], cb_v, sems.at[0]),
        pltpu.async_copy(biases.at[pidx2_v], pb_v, sems.at[1]),
        pltpu.async_copy(labels.at[pl.ds(base, _BPW)], lam_v, sems.at[2]),
        pltpu.async_copy(dots.at[pl.ds(base, _BPW)], dot_v, sems.at[3]),
        pltpu.async_copy(dots.at[pl.ds(_B + base, _BPW)], pvec_v, sems.at[4]),
    ]
    for cp in cps:
        cp.wait()

    lane = lax.iota(jnp.int32, 16)

    acc_w = jnp.zeros((16,), jnp.float32)
    acc_a = jnp.zeros((16,), jnp.float32)
    acc_a2 = jnp.zeros((16,), jnp.float32)
    s_wb = jnp.float32(0.0)
    s_wb2 = jnp.float32(0.0)
    for g in range(_NG):
        lam = lam_v[pl.ds(g * 16, 16)]
        lnl = _ln(lam)
        w = jnp.minimum(jnp.exp(0.75 * (lnl - _LN100)), 1.0)
        acc_w += w
        a16 = cb_v[pl.ds(g * 16, 16)] + pb_v[pl.ds(g * 16, 16)]
        acc_a += a16
        acc_a2 += a16 * a16
        dot16 = dot_v[pl.ds(g * 16, 16)] + pvec_v[pl.ds(g * 16, 16)]
        bv16 = dot16 - lnl
        wb = w * bv16
        s_wb += jnp.sum(wb)
        s_wb2 += jnp.sum(wb * bv16)
    s_w = jnp.sum(acc_w)
    s_a = jnp.sum(acc_a)
    s_a2 = jnp.sum(acc_a2)

    packed = jnp.where(lane == 0, s_w, 0.0)
    packed = jnp.where(lane == 1, s_wb, packed)
    packed = jnp.where(lane == 2, s_wb2, packed)
    packed = jnp.where(lane == 3, s_a, packed)
    packed = jnp.where(lane == 4, s_a2, packed)
    packv_v[...] = packed.astype(jnp.float32)

    pltpu.sync_copy(packv_v, shared.at[pl.ds(sid * 16, 16)])
    plsc.subcore_barrier()

    @pl.when(sid == 0)
    def _():
        pltpu.sync_copy(shared, allv_v)
        tot = allv_v[pl.ds(0, 16)]
        for k in range(1, _NS):
            tot += allv_v[pl.ds(k * 16, 16)]
        outv_v[...] = tot
        pltpu.sync_copy(outv_v, out.at[pl.ds(cid * 16, 16)])


_MESH = plsc.VectorSubcoreMesh(core_axis_name="c", subcore_axis_name="s")
_PARAMS = pltpu.CompilerParams(needs_layout_passes=False)


@jax.jit
def kernel(c_data, p_data, labels, c_table, c_bias, p_table, p_bias):
    dots = pl.kernel(
        _dots_body,
        out_type=jax.ShapeDtypeStruct((_NC * _B,), jnp.float32),
        mesh=_MESH,
        compiler_params=_PARAMS,
        scratch_types=[
            pltpu.VMEM((_B,), jnp.int32),
            pltpu.VMEM((_B,), jnp.int32),
            pltpu.VMEM((_H1,), jnp.float32),
            pltpu.VMEM((_H1,), jnp.float32),
            pltpu.VMEM((_B,), jnp.float32),
            pltpu.VMEM((_B,), jnp.float32),
            pltpu.VMEM((_SEC,), jnp.float32),
            pltpu.VMEM_SHARED((_NS, _B), jnp.float32),
            pltpu.SemaphoreType.DMA((2,)),
        ],
    )(c_data, p_data, c_table.T, p_table.T)

    partials = pl.kernel(
        _loss_body,
        out_type=jax.ShapeDtypeStruct((_NC * 16,), jnp.float32),
        mesh=_MESH,
        compiler_params=_PARAMS,
        scratch_types=[
            pltpu.VMEM((_BPW,), jnp.int32),
            pltpu.VMEM((_BPW,), jnp.int32),
            pltpu.VMEM((_BPW,), jnp.int32),
            pltpu.VMEM((_BPW,), jnp.float32),
            pltpu.VMEM((_BPW,), jnp.float32),
            pltpu.VMEM((_BPW,), jnp.float32),
            pltpu.VMEM((_BPW,), jnp.float32),
            pltpu.VMEM((_BPW,), jnp.float32),
            pltpu.VMEM((_NS * 16,), jnp.float32),
            pltpu.VMEM((16,), jnp.float32),
            pltpu.VMEM((16,), jnp.float32),
            pltpu.VMEM_SHARED((_NS * 16,), jnp.float32),
            pltpu.SemaphoreType.DMA((5,)),
        ],
    )(dots, labels,
      jnp.concatenate([jnp.reshape(c_bias, (-1,)), jnp.reshape(p_bias, (-1,))]),
      c_data, p_data)

    tot = partials[:16] + partials[16:]
    bf = jnp.float32(_B)
    loss = (tot[0] * tot[4] + 2.0 * tot[3] * tot[1] + bf * tot[2]) / (bf * bf)
    return loss


# speedup vs baseline: 5.2712x; 1.0299x over previous
import jax
import jax.numpy as jnp
from jax import lax
from jax.experimental import pallas as pl
from jax.experimental.pallas import tpu as pltpu
from jax.experimental.pallas import tpu_sc as plsc

_B = 4096
_D = 64
_V = 100000
_NC = 2
_NS = 16
_NW = _NC * _NS
_BPW = _B // _NW
_NG = _BPW // 16
_DPW = _D // _NW
_SEC = _B // _NS
_H0 = 49920
_H1 = _V - _H0

_LN2 = 0.6931471805599453
_LN100 = 4.605170185988092
_LOG2_POLY = (
    0.014778755424481588,
    -0.07684890405801897,
    0.1904211707113626,
    -0.32311624947178846,
    0.4724996763418957,
    -0.7203866484224759,
    1.4426521148584406,
    3.1958385927744075e-07,
)


def _ln(x):
    bits = plsc.bitcast(x, jnp.int32)
    e = ((bits >> 23) & 0xFF) - 127
    m = plsc.bitcast((bits & 0x007FFFFF) | 0x3F800000, jnp.float32)
    t = m - 1.0
    p = jnp.full((16,), _LOG2_POLY[0], dtype=jnp.float32)
    for coef in _LOG2_POLY[1:]:
        p = p * t + coef
    return (e.astype(jnp.float32) + p) * _LN2


def _dots_body(c_data, p_data, c_tt, p_tt, out, cidx_v, pidx_v, rowa_v,
               rowb_v, cval_v, dot_v, tot_v, shared, sems):
    cid = lax.axis_index("c")
    sid = lax.axis_index("s")
    wid = cid * _NS + sid

    zero = jnp.zeros((16,), jnp.float32)

    bufs = (rowa_v, rowb_v)
    tasks = []
    for dd in range(_DPW):
        d = wid * _DPW + dd
        for tbl, idxv, kind0, kind1 in ((c_tt, cidx_v, "cw", "ca"),
                                        (p_tt, pidx_v, "p", "p")):
            tasks.append((tbl, d, 0, idxv, kind0))
            tasks.append((tbl, d, 1, idxv, kind1))

    def start(k):
        tbl, d, h, _, _ = tasks[k]
        lo, ln = (0, _H0) if h == 0 else (_H0, _H1)
        return pltpu.async_copy(tbl.at[d, pl.ds(lo, ln)],
                                bufs[k % 2].at[pl.ds(0, ln)],
                                sems.at[k % 2])

    def process(k):
        _, _, h, idxv, kind = tasks[k]
        lo, ln = (0, _H0) if h == 0 else (_H0, _H1)
        buf = bufs[k % 2]

        def body(j, _):
            iv = idxv[pl.ds(j * 16, 16)] - lo
            mask = (iv >= 0) & (iv < ln)
            ivc = jnp.minimum(jnp.maximum(iv, 0), ln - 1)
            g = jnp.where(mask, plsc.load_gather(buf, [ivc]), 0.0)
            if kind == "cw":
                cval_v[pl.ds(j * 16, 16)] = g
            elif kind == "ca":
                cval_v[pl.ds(j * 16, 16)] += g
            else:
                dot_v[pl.ds(j * 16, 16)] += cval_v[pl.ds(j * 16, 16)] * g
            return 0

        lax.fori_loop(0, _B // 16, body, 0)

    cps = {0: start(0), 1: start(1)}
    pltpu.sync_copy(c_data, cidx_v)
    pltpu.sync_copy(p_data, pidx_v)
    for k in range(_B // 16):
        dot_v[pl.ds(k * 16, 16)] = zero
    for k in range(len(tasks)):
        cps[k].wait()
        process(k)
        if k + 2 < len(tasks):
            cps[k + 2] = start(k + 2)

    pltpu.sync_copy(dot_v, shared.at[sid])
    plsc.subcore_barrier()

    base = sid * _SEC
    for k in range(_SEC // 16):
        tot_v[pl.ds(k * 16, 16)] = zero
    for r in range(_NS):
        pltpu.sync_copy(shared.at[r, pl.ds(base, _SEC)], cval_v.at[pl.ds(0, _SEC)])
        for k in range(_SEC // 16):
            tot_v[pl.ds(k * 16, 16)] += cval_v[pl.ds(k * 16, 16)]
    pltpu.sync_copy(tot_v, out.at[pl.ds(cid * _B + base, _SEC)])


def _loss_body(dots, labels, biases, c_data, p_data, out, cidx_v, pidx_v,
               pidx2_v, lam_v, dot_v, cb_v, pb_v, pvec_v, allv_v, packv_v,
               outv_v, shared, sems):
    cid = lax.axis_index("c")
    sid = lax.axis_index("s")
    wid = cid * _NS + sid
    base = wid * _BPW

    pltpu.sync_copy(c_data.at[pl.ds(base, _BPW)], cidx_v)
    pltpu.sync_copy(p_data.at[pl.ds(base, _BPW)], pidx_v)
    for g in range(_NG):
        pidx2_v[pl.ds(g * 16, 16)] = pidx_v[pl.ds(g * 16, 16)] + _V
    cps = [
        pltpu.async_copy(biases.at[cidx_v], cb_v, sems.at[0]),
        pltpu.async_copy(biases.at[pidx2_v], pb_v, sems.at[1]),
        pltpu.async_copy(labels.at[pl.ds(base, _BPW)], lam_v, sems.at[2]),
        pltpu.async_copy(dots.at[pl.ds(base, _BPW)], dot_v, sems.at[3]),
        pltpu.async_copy(dots.at[pl.ds(_B + base, _BPW)], pvec_v, sems.at[4]),
    ]
    for cp in cps:
        cp.wait()

    lane = lax.iota(jnp.int32, 16)

    acc_w = jnp.zeros((16,), jnp.float32)
    acc_a = jnp.zeros((16,), jnp.float32)
    acc_a2 = jnp.zeros((16,), jnp.float32)
    s_wb = jnp.float32(0.0)
    s_wb2 = jnp.float32(0.0)
    for g in range(_NG):
        lam = lam_v[pl.ds(g * 16, 16)]
        lnl = _ln(lam)
        w = jnp.minimum(jnp.exp(0.75 * (lnl - _LN100)), 1.0)
        acc_w += w
        a16 = cb_v[pl.ds(g * 16, 16)] + pb_v[pl.ds(g * 16, 16)]
        acc_a += a16
        acc_a2 += a16 * a16
        dot16 = dot_v[pl.ds(g * 16, 16)] + pvec_v[pl.ds(g * 16, 16)]
        bv16 = dot16 - lnl
        wb = w * bv16
        s_wb += jnp.sum(wb)
        s_wb2 += jnp.sum(wb * bv16)
    s_w = jnp.sum(acc_w)
    s_a = jnp.sum(acc_a)
    s_a2 = jnp.sum(acc_a2)

    packed = jnp.where(lane == 0, s_w, 0.0)
    packed = jnp.where(lane == 1, s_wb, packed)
    packed = jnp.where(lane == 2, s_wb2, packed)
    packed = jnp.where(lane == 3, s_a, packed)
    packed = jnp.where(lane == 4, s_a2, packed)
    packv_v[...] = packed.astype(jnp.float32)

    pltpu.sync_copy(packv_v, shared.at[pl.ds(sid * 16, 16)])
    plsc.subcore_barrier()

    @pl.when(sid == 0)
    def _():
        pltpu.sync_copy(shared, allv_v)
        tot = allv_v[pl.ds(0, 16)]
        for k in range(1, _NS):
            tot += allv_v[pl.ds(k * 16, 16)]
        outv_v[...] = tot
        pltpu.sync_copy(outv_v, out.at[pl.ds(cid * 16, 16)])


_MESH = plsc.VectorSubcoreMesh(core_axis_name="c", subcore_axis_name="s")
_PARAMS = pltpu.CompilerParams(needs_layout_passes=False)


@jax.jit
def kernel(c_data, p_data, labels, c_table, c_bias, p_table, p_bias):
    dots = pl.kernel(
        _dots_body,
        out_type=jax.ShapeDtypeStruct((_NC * _B,), jnp.float32),
        mesh=_MESH,
        compiler_params=_PARAMS,
        scratch_types=[
            pltpu.VMEM((_B,), jnp.int32),
            pltpu.VMEM((_B,), jnp.int32),
            pltpu.VMEM((_H1,), jnp.float32),
            pltpu.VMEM((_H1,), jnp.float32),
            pltpu.VMEM((_B,), jnp.float32),
            pltpu.VMEM((_B,), jnp.float32),
            pltpu.VMEM((_SEC,), jnp.float32),
            pltpu.VMEM_SHARED((_NS, _B), jnp.float32),
            pltpu.SemaphoreType.DMA((2,)),
        ],
    )(c_data, p_data, c_table.T, p_table.T)

    partials = pl.kernel(
        _loss_body,
        out_type=jax.ShapeDtypeStruct((_NC * 16,), jnp.float32),
        mesh=_MESH,
        compiler_params=_PARAMS,
        scratch_types=[
            pltpu.VMEM((_BPW,), jnp.int32),
            pltpu.VMEM((_BPW,), jnp.int32),
            pltpu.VMEM((_BPW,), jnp.int32),
            pltpu.VMEM((_BPW,), jnp.float32),
            pltpu.VMEM((_BPW,), jnp.float32),
            pltpu.VMEM((_BPW,), jnp.float32),
            pltpu.VMEM((_BPW,), jnp.float32),
            pltpu.VMEM((_BPW,), jnp.float32),
            pltpu.VMEM((_NS * 16,), jnp.float32),
            pltpu.VMEM((16,), jnp.float32),
            pltpu.VMEM((16,), jnp.float32),
            pltpu.VMEM_SHARED((_NS * 16,), jnp.float32),
            pltpu.SemaphoreType.DMA((5,)),
        ],
    )(dots, labels,
      jnp.concatenate([jnp.reshape(c_bias, (-1,)), jnp.reshape(p_bias, (-1,))]),
      c_data, p_data)

    tot = partials[:16] + partials[16:]
    bf = jnp.float32(_B)
    loss = (tot[0] * tot[4] + 2.0 * tot[3] * tot[1] + bf * tot[2]) / (bf * bf)
    return loss
